# Initial kernel scaffold; baseline (speedup 1.0000x reference)
import jax
import jax.numpy as jnp
from jax.experimental import pallas as pl


def kernel(query, responses, gate_W, gate_b):
    raise NotImplementedError("write your pallas kernel here")


# speedup vs baseline: 1.1015x; 1.1015x over previous
import functools
import jax
import jax.numpy as jnp
from jax.experimental import pallas as pl
from jax.experimental.pallas import tpu as pltpu

_N_UIDS = 8192
_TOPK = 20
_UID_BLK = 1024
_ROW_BLK = 128


def _gate_body(q_ref, w_ref, b_ref, o_ref):
    mq = jnp.mean(q_ref[...], axis=0)
    prod = w_ref[...] * mq[None, None, :]
    o_ref[...] = jnp.sum(prod, axis=2) + b_ref[...]


def _gate_stage(query, gate_W, gate_b):
    nblk = _N_UIDS // _UID_BLK
    w3 = gate_W.reshape(_N_UIDS // 128, 128, gate_W.shape[1])
    b2 = gate_b.reshape(_N_UIDS // 128, 128)
    out = pl.pallas_call(
        _gate_body,
        grid=(nblk,),
        in_specs=[
            pl.BlockSpec(query.shape, lambda i: (0, 0)),
            pl.BlockSpec((_UID_BLK // 128, 128, gate_W.shape[1]),
                         lambda i: (i, 0, 0)),
            pl.BlockSpec((_UID_BLK // 128, 128), lambda i: (i, 0)),
        ],
        out_specs=pl.BlockSpec((_UID_BLK // 128, 128), lambda i: (i, 0)),
        out_shape=jax.ShapeDtypeStruct((_N_UIDS // 128, 128), jnp.float32),
    )(query, w3, b2)
    return out


def _topk_body(mw_ref, tw_ref, ow_ref, rs_ref):
    vals = mw_ref[...]
    ridx = jax.lax.broadcasted_iota(jnp.int32, vals.shape, 0)
    cidx = jax.lax.broadcasted_iota(jnp.int32, vals.shape, 1)
    flat = ridx * 128 + cidx
    big = jnp.int32(2 ** 30)
    tvals, tidxs = [], []
    for _ in range(_TOPK):
        m = jnp.max(vals)
        i = jnp.min(jnp.where(vals == m, flat, big))
        tvals.append(m)
        tidxs.append(i)
        vals = jnp.where(flat == i, -jnp.inf, vals)

    lane = jax.lax.broadcasted_iota(jnp.int32, (1, 128), 1)
    tw = jnp.zeros((1, 128), jnp.float32)
    for r in range(_TOPK):
        tw = jnp.where(lane == r, tvals[r], tw)
    tw_ref[...] = tw

    mn = tvals[-1]
    total = tvals[0] - mn
    for r in range(1, _TOPK):
        total = total + (tvals[r] - mn)
    ow = jnp.zeros(vals.shape, jnp.float32)
    member = jnp.zeros(vals.shape, jnp.bool_)
    for r in range(_TOPK):
        hit = flat == tidxs[r]
        ow = jnp.where(hit, (tvals[r] - mn) / total, ow)
        member = jnp.logical_or(member, hit)
    ow_ref[...] = ow
    rs_ref[...] = jnp.where(member, jnp.float32(32.0), jnp.float32(0.0))


def _topk_stage(mw):
    tw, ow, rs = pl.pallas_call(
        _topk_body,
        out_shape=[
            jax.ShapeDtypeStruct((1, 128), jnp.float32),
            jax.ShapeDtypeStruct(mw.shape, jnp.float32),
            jax.ShapeDtypeStruct(mw.shape, jnp.float32),
        ],
    )(mw)
    return tw, ow, rs


def _join_body(w_ref, r_ref, o_ref):
    acc = r_ref[0] * w_ref[0]
    for i in range(1, _TOPK):
        acc = acc + r_ref[i] * w_ref[i]
    o_ref[...] = acc


def _join_stage(tw20, responses):
    k, b, s, d = responses.shape
    rows = b * s
    r3 = responses.reshape(k, rows, d)
    out = pl.pallas_call(
        _join_body,
        grid=(rows // _ROW_BLK,),
        in_specs=[
            pl.BlockSpec(memory_space=pltpu.SMEM),
            pl.BlockSpec((k, _ROW_BLK, d), lambda i: (0, i, 0)),
        ],
        out_specs=pl.BlockSpec((_ROW_BLK, d), lambda i: (i, 0)),
        out_shape=jax.ShapeDtypeStruct((rows, d), jnp.float32),
    )(tw20, r3)
    return out.reshape(b, s, d)


def kernel(query, responses, gate_W, gate_b):
    mw = _gate_stage(query, gate_W, gate_b)
    tw, ow, rs = _topk_stage(mw)
    weighted = _join_stage(tw[0, :_TOPK], responses)
    return weighted, ow.reshape(_N_UIDS), rs.reshape(_N_UIDS)
